# baseline (speedup 1.0000x reference)
import math

import jax
import jax.numpy as jnp
from jax.experimental import pallas as pl
from jax.experimental.pallas import tpu as pltpu

WINDOW = 15
S = 512
D = 256
NSPK = 9
T = 128
NT = S // T
SLAB = 3 * T

_ACOS_C = (
    1.5707288 / math.pi,
    -0.2121144 / math.pi,
    0.0742610 / math.pi,
    -0.0187293 / math.pi,
)


def _wfun(cos):
    ax = jnp.abs(cos)
    p = jnp.float32(_ACOS_C[3])
    for c in _ACOS_C[2::-1]:
        p = p * ax + jnp.float32(c)
    r = jnp.sqrt(jnp.maximum(1.0 - ax, 0.0)) * p
    return jnp.where(cos >= 0.0, 1.0 - r, r)


def _slab_lo(k):
    return jnp.minimum(jnp.maximum((k - 1) * T, 0), S - SLAB)


def _adj_kernel(
    dia_ref,
    x_ref,
    q_ref,
    out_ref,
    xn_ref,
    spkr_ref,
    spkc_ref,
    dinvr_ref,
    dinvc_ref,
    pre_ref,
):
    s = pl.program_id(0)
    nsteps = pl.num_programs(0)

    @pl.when(s == 0)
    def _init():
        dinvr_ref[...] = jnp.ones((1, S), jnp.float32)

    @pl.when(s < nsteps - 1)
    def _compute():
        b = s // NT
        k = s % NT
        dl = dia_ref[b]

        @pl.when(k == 0)
        def _per_batch():
            xb = x_ref[0]
            xn_ref[...] = xb * jax.lax.rsqrt(
                jnp.maximum(jnp.sum(xb * xb, axis=1, keepdims=True), 1e-16)
            )
            q = q_ref[0]
            qmax = jnp.max(q, axis=0)
            io = jax.lax.broadcasted_iota(jnp.int32, (16, S), 0)
            spk = jnp.min(jnp.where(q >= qmax[None, :], io, 16), axis=0)
            spkr_ref[...] = spk[None, :]
            spkc_ref[...] = spk[:, None]

        r0 = pl.multiple_of(k * T, T)
        lo = pl.multiple_of(_slab_lo(k), T)
        xr = xn_ref[pl.ds(r0, T), :]
        xc = xn_ref[pl.ds(lo, SLAB), :]
        cos = jax.lax.dot_general(
            xr, xc, (((1,), (1,)), ((), ())), preferred_element_type=jnp.float32
        )
        w = _wfun(cos)
        ii = jax.lax.broadcasted_iota(jnp.int32, (T, SLAB), 0) + r0
        jj = jax.lax.broadcasted_iota(jnp.int32, (T, SLAB), 1) + lo
        winm = (jnp.abs(ii - jj) <= WINDOW) & (ii < dl) & (jj < dl)
        samet = spkc_ref[pl.ds(r0, T), :] == spkr_ref[:, pl.ds(lo, SLAB)]
        spkf = (winm & samet).astype(jnp.float32)
        winf = winm.astype(jnp.float32)
        cnt = jnp.sum(spkf, axis=1)
        gate = (cnt > 1.0).astype(jnp.float32)[:, None]
        pre = w * (winf + spkf * gate)
        pre_ref[:, pl.ds(pl.multiple_of((s % 2) * SLAB, SLAB), SLAB)] = pre
        deg = jnp.sum(pre, axis=1)
        dv = jax.lax.rsqrt(jnp.where(deg == 0.0, 1.0, deg))
        dinvr_ref[:, pl.ds(r0, T)] = dv[None, :]
        dinvc_ref[:, pl.ds(pl.multiple_of((s % 2) * 128, 128), 1)] = dv[:, None]

    @pl.when(s >= 1)
    def _write():
        pk = (s - 1) % NT
        plo = pl.multiple_of(_slab_lo(pk), T)
        dr = dinvc_ref[:, pl.ds(pl.multiple_of(((s - 1) % 2) * 128, 128), 1)]
        dc = dinvr_ref[:, pl.ds(plo, SLAB)]
        scaled = pre_ref[:, pl.ds(pl.multiple_of(((s - 1) % 2) * SLAB, SLAB), SLAB)] * dr * dc
        out_ref[0] = jnp.zeros((T, S), jnp.float32)
        out_ref[0, :, pl.ds(plo, SLAB)] = scaled


def kernel(x, dia_len, qmask):
    B = x.shape[0]
    G = B * NT + 1
    qt = jnp.transpose(qmask, (1, 2, 0))
    qt = jnp.concatenate(
        [qt, jnp.full((B, 16 - NSPK, S), -1.0, jnp.float32)], axis=1
    )
    dl = dia_len.astype(jnp.int32)
    grid_spec = pltpu.PrefetchScalarGridSpec(
        num_scalar_prefetch=1,
        grid=(G,),
        in_specs=[
            pl.BlockSpec(
                (1, S, D), lambda s, d: (jnp.minimum(s, G - 2) // NT, 0, 0)
            ),
            pl.BlockSpec(
                (1, 16, S), lambda s, d: (jnp.minimum(s, G - 2) // NT, 0, 0)
            ),
        ],
        out_specs=pl.BlockSpec(
            (1, T, S),
            lambda s, d: (
                jnp.maximum(s - 1, 0) // NT,
                jnp.maximum(s - 1, 0) % NT,
                0,
            ),
        ),
        scratch_shapes=[
            pltpu.VMEM((S, D), jnp.float32),
            pltpu.VMEM((1, S), jnp.int32),
            pltpu.VMEM((S, 1), jnp.int32),
            pltpu.VMEM((1, S), jnp.float32),
            pltpu.VMEM((T, 256), jnp.float32),
            pltpu.VMEM((T, 2 * SLAB), jnp.float32),
        ],
    )
    return pl.pallas_call(
        _adj_kernel,
        grid_spec=grid_spec,
        out_shape=jax.ShapeDtypeStruct((B, S, S), jnp.float32),
    )(dl, x, qt)


# speedup vs baseline: 2.1947x; 2.1947x over previous
import math

import jax
import jax.numpy as jnp
from jax.experimental import pallas as pl
from jax.experimental.pallas import tpu as pltpu

WINDOW = 15
S = 512
D = 256
NSPK = 9
T = 128
NT = S // T

_ACOS_C = (
    1.5707288 / math.pi,
    -0.2121144 / math.pi,
    0.0742610 / math.pi,
    -0.0187293 / math.pi,
)


def _wfun(cos):
    ax = jnp.abs(cos)
    p = jnp.float32(_ACOS_C[3])
    for c in _ACOS_C[2::-1]:
        p = p * ax + jnp.float32(c)
    r = jnp.sqrt(jnp.maximum(1.0 - ax, 0.0)) * p
    return jnp.where(cos >= 0.0, 1.0 - r, r)


def _adj_kernel(dia_ref, x_ref, q_ref, out_ref):
    b = pl.program_id(0)
    dl = dia_ref[b]
    xb = x_ref[0]
    xn = xb * jax.lax.rsqrt(
        jnp.maximum(jnp.sum(xb * xb, axis=1, keepdims=True), 1e-16)
    )

    q = q_ref[0]
    qmax = jnp.max(q, axis=0)
    io = jax.lax.broadcasted_iota(jnp.int32, (16, S), 0)
    spk = jnp.min(jnp.where(q >= qmax[None, :], io, 16), axis=0)

    ii0 = jax.lax.broadcasted_iota(jnp.int32, (T, T), 0)
    jj0 = jax.lax.broadcasted_iota(jnp.int32, (T, T), 1)
    band_mask = {
        ofs: jnp.abs(ii0 - (jj0 + ofs)) <= WINDOW for ofs in (-T, 0, T)
    }
    vcol = jax.lax.broadcasted_iota(jnp.int32, (S, 1), 0) < dl
    vrow = jax.lax.broadcasted_iota(jnp.int32, (1, S), 1) < dl

    dinv_parts = []
    prev_tiles = None
    for ti in range(NT):
        r0 = ti * T
        xr = xn[r0 : r0 + T]
        spk_r = spk[r0 : r0 + T]
        valid_r = vcol[r0 : r0 + T, :]
        tjs = [tj for tj in (ti - 1, ti, ti + 1) if 0 <= tj < NT]
        tiles = []
        spk_sum = None
        for tj in tjs:
            c0 = tj * T
            cos = jax.lax.dot_general(
                xr,
                xn[c0 : c0 + T],
                (((1,), (1,)), ((), ())),
                preferred_element_type=jnp.float32,
            )
            w = _wfun(cos)
            winm = (
                band_mask[c0 - r0]
                & valid_r
                & vrow[:, c0 : c0 + T]
            )
            samet = spk_r[:, None] == spk[c0 : c0 + T][None, :]
            spkf = (winm & samet).astype(jnp.float32)
            winf = winm.astype(jnp.float32)
            spk_sum = spkf if spk_sum is None else spk_sum + spkf
            tiles.append((c0, w, winf, spkf))
        cnt = jnp.sum(spk_sum, axis=1)
        gate = (cnt > 1.0).astype(jnp.float32)[:, None]
        pre_sum = None
        pres = []
        for c0, w, winf, spkf in tiles:
            pre = w * (winf + spkf * gate)
            pre_sum = pre if pre_sum is None else pre_sum + pre
            pres.append((c0, pre))
        deg = jnp.sum(pre_sum, axis=1)
        lo = tjs[0] * T
        hi = (tjs[-1] + 1) * T
        if lo > 0:
            out_ref[0, r0 : r0 + T, 0:lo] = jnp.zeros((T, lo), jnp.float32)
        if hi < S:
            out_ref[0, r0 : r0 + T, hi:S] = jnp.zeros((T, S - hi), jnp.float32)
        dinv_parts.append(jax.lax.rsqrt(jnp.where(deg == 0.0, 1.0, deg)))

        if prev_tiles is not None:
            p0 = (ti - 1) * T
            dr = dinv_parts[ti - 1][:, None]
            for c0, pre in prev_tiles:
                dc = dinv_parts[c0 // T][None, :]
                out_ref[0, p0 : p0 + T, c0 : c0 + T] = pre * dr * dc
        prev_tiles = pres

    p0 = (NT - 1) * T
    dr = dinv_parts[NT - 1][:, None]
    for c0, pre in prev_tiles:
        dc = dinv_parts[c0 // T][None, :]
        out_ref[0, p0 : p0 + T, c0 : c0 + T] = pre * dr * dc


def kernel(x, dia_len, qmask):
    B = x.shape[0]
    qt = jnp.transpose(qmask, (1, 2, 0))
    qt = jnp.concatenate(
        [qt, jnp.full((B, 16 - NSPK, S), -1.0, jnp.float32)], axis=1
    )
    dl = dia_len.astype(jnp.int32)
    grid_spec = pltpu.PrefetchScalarGridSpec(
        num_scalar_prefetch=1,
        grid=(B,),
        in_specs=[
            pl.BlockSpec((1, S, D), lambda b, d: (b, 0, 0)),
            pl.BlockSpec((1, 16, S), lambda b, d: (b, 0, 0)),
        ],
        out_specs=pl.BlockSpec((1, S, S), lambda b, d: (b, 0, 0)),
    )
    return pl.pallas_call(
        _adj_kernel,
        grid_spec=grid_spec,
        out_shape=jax.ShapeDtypeStruct((B, S, S), jnp.float32),
        compiler_params=pltpu.CompilerParams(
            dimension_semantics=("parallel",)
        ),
    )(dl, x, qt)
